# Initial kernel scaffold; baseline (speedup 1.0000x reference)
import jax
import jax.numpy as jnp
from jax.experimental import pallas as pl


def kernel(boxes, scores):
    raise NotImplementedError("write your pallas kernel here")


# speedup vs baseline: 121.8494x; 121.8494x over previous
import jax
import jax.numpy as jnp
from jax.experimental import pallas as pl
from jax.experimental.pallas import tpu as pltpu

_PRE = 2000
_POST = 1000
_PAD = 2048
_OUTPAD = 1024
_THRESH = 0.7
_BLK = 128
_NBLK = _PAD // _BLK


def _nms_kernel(b_ref, bt_ref, d_ref, o_ref, m_ref):
    f32 = jnp.float32
    x1r = bt_ref[0:1, :]
    y1r = bt_ref[1:2, :]
    x2r = bt_ref[2:3, :]
    y2r = bt_ref[3:4, :]
    area_r = (x2r - x1r) * (y2r - y1r)
    jidx = jax.lax.broadcasted_iota(jnp.int32, (1, _PAD), 1)

    def row_block(blk, carry):
        sl = pl.ds(blk * _BLK, _BLK)
        x1c = b_ref[sl, 0:1]
        y1c = b_ref[sl, 1:2]
        x2c = b_ref[sl, 2:3]
        y2c = b_ref[sl, 3:4]
        area_c = (x2c - x1c) * (y2c - y1c)
        wx = jnp.maximum(jnp.minimum(x2c, x2r) - jnp.maximum(x1c, x1r), 0.0)
        wy = jnp.maximum(jnp.minimum(y2c, y2r) - jnp.maximum(y1c, y1r), 0.0)
        inter = wx * wy
        union = area_c + area_r - inter
        iou = inter / jnp.maximum(union, 1e-9)
        iidx = blk * _BLK + jax.lax.broadcasted_iota(jnp.int32, (_BLK, 1), 0)
        m = ((iou > _THRESH) & (iidx < jidx)).astype(jnp.bfloat16)
        m_ref[sl, :] = m
        return carry

    jax.lax.fori_loop(0, _NBLK, row_block, 0)

    def cond(c):
        return c[1]

    def body(c):
        k, _ = c
        sup = jax.lax.dot_general(
            k.astype(jnp.bfloat16),
            m_ref[...],
            (((1,), (0,)), ((), ())),
            preferred_element_type=f32,
        )
        nk = (sup <= 0.0).astype(f32)
        changed = jnp.any(nk != k)
        return (nk, changed)

    k0 = jnp.ones((1, _PAD), dtype=f32)
    k, _ = jax.lax.while_loop(cond, body, (k0, jnp.bool_(True)))

    k = k * (jidx < _PRE).astype(f32)

    def l_block(blk, carry):
        sl = pl.ds(blk * _BLK, _BLK)
        iidx = blk * _BLK + jax.lax.broadcasted_iota(jnp.int32, (_BLK, 1), 0)
        m_ref[sl, :] = (iidx <= jidx).astype(jnp.bfloat16)
        return carry

    jax.lax.fori_loop(0, _NBLK, l_block, 0)

    csum = jax.lax.dot_general(
        k.astype(jnp.bfloat16),
        m_ref[...],
        (((1,), (0,)), ((), ())),
        preferred_element_type=f32,
    )
    n_kept = csum[0:1, _PAD - 1 : _PAD]
    csup = (jidx + 1).astype(f32) - csum
    dest = jnp.where(k > 0.0, csum - 1.0, n_kept + csup - 1.0)
    dest_i = dest.astype(jnp.int32)

    def o_block(blk, carry):
        ridx = blk * _BLK + jax.lax.broadcasted_iota(jnp.int32, (_BLK, 1), 0)
        p = (dest_i == ridx).astype(f32)
        o_ref[pl.ds(blk * _BLK, _BLK), :] = jax.lax.dot_general(
            p,
            d_ref[...],
            (((1,), (0,)), ((), ())),
            preferred_element_type=f32,
        )
        return carry

    jax.lax.fori_loop(0, _OUTPAD // _BLK, o_block, 0)


def kernel(boxes, scores):
    f32 = jnp.float32
    w = boxes[:, 2] - boxes[:, 0]
    h = boxes[:, 3] - boxes[:, 1]
    valid = (w >= 0.0) & (h >= 0.0)
    scores_f = jnp.where(valid, scores, -jnp.inf)

    top_scores, top_idx = jax.lax.top_k(scores_f, _PRE)
    b = boxes[top_idx]

    pad = _PAD - _PRE
    b_p = jnp.pad(b, ((0, pad), (0, 0)))
    s_p = jnp.pad(top_scores, (0, pad))
    bt = b_p.T
    d = jnp.concatenate(
        [b_p, s_p[:, None], jnp.zeros((_PAD, 3), f32)], axis=1
    )

    out = pl.pallas_call(
        _nms_kernel,
        out_shape=jax.ShapeDtypeStruct((_OUTPAD, 8), f32),
        scratch_shapes=[pltpu.VMEM((_PAD, _PAD), jnp.bfloat16)],
    )(b_p, bt, d)
    return out[:_POST, :5]


# speedup vs baseline: 136.6302x; 1.1213x over previous
import jax
import jax.numpy as jnp
from jax.experimental import pallas as pl
from jax.experimental.pallas import tpu as pltpu

_PRE = 2000
_POST = 1000
_PAD = 2048
_OUTPAD = 1024
_THRESH = 0.7
_BLK = 128
_NBLK = _PAD // _BLK


def _nms_kernel(b_ref, bt_ref, d_ref, o_ref, mcol_ref, kv_ref, cs_ref):
    f32 = jnp.float32
    bf16 = jnp.bfloat16

    x1i = b_ref[:, 0:1]
    y1i = b_ref[:, 1:2]
    x2i = b_ref[:, 2:3]
    y2i = b_ref[:, 3:4]
    area_i = (x2i - x1i) * (y2i - y1i)
    jidx = jax.lax.broadcasted_iota(jnp.int32, (1, _PAD), 1)

    kv_ref[...] = jnp.ones((1, _PAD), f32)

    for b in range(_NBLK):
        c0 = b * _BLK
        h = c0 + _BLK
        x1j = bt_ref[0:1, c0:h]
        y1j = bt_ref[1:2, c0:h]
        x2j = bt_ref[2:3, c0:h]
        y2j = bt_ref[3:4, c0:h]
        area_j = (x2j - x1j) * (y2j - y1j)
        wx = jnp.maximum(
            jnp.minimum(x2i[0:h], x2j) - jnp.maximum(x1i[0:h], x1j), 0.0
        )
        wy = jnp.maximum(
            jnp.minimum(y2i[0:h], y2j) - jnp.maximum(y1i[0:h], y1j), 0.0
        )
        inter = wx * wy
        union = area_i[0:h] + area_j - inter
        iou = inter / jnp.maximum(union, 1e-9)
        iidx = jax.lax.broadcasted_iota(jnp.int32, (h, 1), 0)
        jidx_b = c0 + jax.lax.broadcasted_iota(jnp.int32, (1, _BLK), 1)
        mcol_ref[0:h, :] = ((iou > _THRESH) & (iidx < jidx_b)).astype(bf16)

        if b == 0:
            allowed = jnp.ones((1, _BLK), f32)
        else:
            kp = kv_ref[0:1, 0:c0].astype(bf16)
            supb = jax.lax.dot_general(
                kp,
                mcol_ref[0:c0, :],
                (((1,), (0,)), ((), ())),
                preferred_element_type=f32,
            )
            allowed = (supb <= 0.0).astype(f32)

        mbb = mcol_ref[c0:h, :]

        def cond(c):
            return c[1]

        def body(c, allowed=allowed, mbb=mbb):
            kb, _ = c
            sl = jax.lax.dot_general(
                kb.astype(bf16),
                mbb,
                (((1,), (0,)), ((), ())),
                preferred_element_type=f32,
            )
            nkb = allowed * (sl <= 0.0).astype(f32)
            return (nkb, jnp.any(nkb != kb))

        kb, _ = jax.lax.while_loop(cond, body, (allowed, jnp.bool_(True)))
        kv_ref[0:1, c0:h] = kb

    k = kv_ref[...] * (jidx < _PRE).astype(f32)
    kv_ref[...] = k

    u_tri = (
        jax.lax.broadcasted_iota(jnp.int32, (_BLK, _BLK), 0)
        <= jax.lax.broadcasted_iota(jnp.int32, (_BLK, _BLK), 1)
    ).astype(bf16)
    carry = jnp.zeros((1, 1), f32)
    for b in range(_NBLK):
        c0 = b * _BLK
        kb = kv_ref[0:1, c0 : c0 + _BLK].astype(bf16)
        cb = (
            jax.lax.dot_general(
                kb,
                u_tri,
                (((1,), (0,)), ((), ())),
                preferred_element_type=f32,
            )
            + carry
        )
        cs_ref[0:1, c0 : c0 + _BLK] = cb
        carry = cb[0:1, _BLK - 1 : _BLK]

    csum = cs_ref[...]
    n_kept = cs_ref[0:1, _PAD - 1 : _PAD]
    csup = (jidx + 1).astype(f32) - csum
    dest = jnp.where(k > 0.0, csum - 1.0, n_kept + csup - 1.0)
    dest_i = dest.astype(jnp.int32)

    for ob in range(_OUTPAD // _BLK):
        ridx = ob * _BLK + jax.lax.broadcasted_iota(jnp.int32, (_BLK, 1), 0)
        p = (dest_i == ridx).astype(f32)
        o_ref[ob * _BLK : (ob + 1) * _BLK, :] = jax.lax.dot_general(
            p,
            d_ref[...],
            (((1,), (0,)), ((), ())),
            preferred_element_type=f32,
        )


def kernel(boxes, scores):
    f32 = jnp.float32
    w = boxes[:, 2] - boxes[:, 0]
    h = boxes[:, 3] - boxes[:, 1]
    valid = (w >= 0.0) & (h >= 0.0)
    scores_f = jnp.where(valid, scores, -jnp.inf)

    top_scores, top_idx = jax.lax.top_k(scores_f, _PRE)
    b = boxes[top_idx]

    pad = _PAD - _PRE
    b_p = jnp.pad(b, ((0, pad), (0, 0)))
    s_p = jnp.pad(top_scores, (0, pad))
    bt = b_p.T
    d = jnp.concatenate(
        [b_p, s_p[:, None], jnp.zeros((_PAD, 3), f32)], axis=1
    )

    out = pl.pallas_call(
        _nms_kernel,
        out_shape=jax.ShapeDtypeStruct((_OUTPAD, 8), f32),
        scratch_shapes=[
            pltpu.VMEM((_PAD, _BLK), jnp.bfloat16),
            pltpu.VMEM((1, _PAD), f32),
            pltpu.VMEM((1, _PAD), f32),
        ],
    )(b_p, bt, d)
    return out[:_POST, :5]


# speedup vs baseline: 140.1674x; 1.0259x over previous
import jax
import jax.numpy as jnp
from jax.experimental import pallas as pl
from jax.experimental.pallas import tpu as pltpu

_PRE = 2000
_POST = 1000
_PAD = 2048
_OUTPAD = 1024
_THRESH = 0.7
_BLK = 128
_NBLK = _PAD // _BLK


def _nms_kernel(bt_ref, d_ref, o_ref, mcol_ref, kv_ref, cs_ref):
    f32 = jnp.float32
    bf16 = jnp.bfloat16

    x1i = d_ref[:, 0:1]
    y1i = d_ref[:, 1:2]
    x2i = d_ref[:, 2:3]
    y2i = d_ref[:, 3:4]
    area_i = (x2i - x1i) * (y2i - y1i)
    jidx = jax.lax.broadcasted_iota(jnp.int32, (1, _PAD), 1)

    kv_ref[...] = jnp.ones((1, _PAD), f32)

    for b in range(_NBLK):
        c0 = b * _BLK
        h = c0 + _BLK
        x1j = bt_ref[0:1, c0:h]
        y1j = bt_ref[1:2, c0:h]
        x2j = bt_ref[2:3, c0:h]
        y2j = bt_ref[3:4, c0:h]
        area_j = (x2j - x1j) * (y2j - y1j)
        wx = jnp.maximum(
            jnp.minimum(x2i[0:h], x2j) - jnp.maximum(x1i[0:h], x1j), 0.0
        )
        wy = jnp.maximum(
            jnp.minimum(y2i[0:h], y2j) - jnp.maximum(y1i[0:h], y1j), 0.0
        )
        inter = wx * wy
        union = area_i[0:h] + area_j - inter
        iou = inter / jnp.maximum(union, 1e-9)
        iidx = jax.lax.broadcasted_iota(jnp.int32, (h, 1), 0)
        jidx_b = c0 + jax.lax.broadcasted_iota(jnp.int32, (1, _BLK), 1)
        mcol_ref[0:h, :] = ((iou > _THRESH) & (iidx < jidx_b)).astype(bf16)

        if b == 0:
            allowed = jnp.ones((1, _BLK), f32)
        else:
            kp = kv_ref[0:1, 0:c0].astype(bf16)
            supb = jax.lax.dot_general(
                kp,
                mcol_ref[0:c0, :],
                (((1,), (0,)), ((), ())),
                preferred_element_type=f32,
            )
            allowed = (supb <= 0.0).astype(f32)

        mbb = mcol_ref[c0:h, :]

        def cond(c):
            return c[1]

        def body(c, allowed=allowed, mbb=mbb):
            kb, _ = c
            sl = jax.lax.dot_general(
                kb.astype(bf16),
                mbb,
                (((1,), (0,)), ((), ())),
                preferred_element_type=f32,
            )
            nkb = allowed * (sl <= 0.0).astype(f32)
            return (nkb, jnp.any(nkb != kb))

        kb, _ = jax.lax.while_loop(cond, body, (allowed, jnp.bool_(True)))
        kv_ref[0:1, c0:h] = kb

    k = kv_ref[...] * (jidx < _PRE).astype(f32)
    kv_ref[...] = k

    u_tri = (
        jax.lax.broadcasted_iota(jnp.int32, (_BLK, _BLK), 0)
        <= jax.lax.broadcasted_iota(jnp.int32, (_BLK, _BLK), 1)
    ).astype(bf16)
    carry = jnp.zeros((1, 1), f32)
    for b in range(_NBLK):
        c0 = b * _BLK
        kb = kv_ref[0:1, c0 : c0 + _BLK].astype(bf16)
        cb = (
            jax.lax.dot_general(
                kb,
                u_tri,
                (((1,), (0,)), ((), ())),
                preferred_element_type=f32,
            )
            + carry
        )
        cs_ref[0:1, c0 : c0 + _BLK] = cb
        carry = cb[0:1, _BLK - 1 : _BLK]

    csum = cs_ref[...]
    n_kept = cs_ref[0:1, _PAD - 1 : _PAD]
    csup = (jidx + 1).astype(f32) - csum
    dest = jnp.where(k > 0.0, csum - 1.0, n_kept + csup - 1.0)
    dest_i = dest.astype(jnp.int32)

    for ob in range(_OUTPAD // _BLK):
        ridx = ob * _BLK + jax.lax.broadcasted_iota(jnp.int32, (_BLK, 1), 0)
        p = (dest_i == ridx).astype(f32)
        o_ref[ob * _BLK : (ob + 1) * _BLK, :] = jax.lax.dot_general(
            p,
            d_ref[...],
            (((1,), (0,)), ((), ())),
            preferred_element_type=f32,
        )


def kernel(boxes, scores):
    f32 = jnp.float32
    w = boxes[:, 2] - boxes[:, 0]
    h = boxes[:, 3] - boxes[:, 1]
    valid = (w >= 0.0) & (h >= 0.0)
    scores_f = jnp.where(valid, scores, -jnp.inf)

    top_scores, top_idx = jax.lax.top_k(scores_f, _PRE)
    b = boxes[top_idx]

    pad = _PAD - _PRE
    b_p = jnp.pad(b, ((0, pad), (0, 0)))
    s_p = jnp.pad(top_scores, (0, pad))
    bt = b_p.T
    d = jnp.concatenate(
        [b_p, s_p[:, None], jnp.zeros((_PAD, 3), f32)], axis=1
    )

    out = pl.pallas_call(
        _nms_kernel,
        out_shape=jax.ShapeDtypeStruct((_OUTPAD, 8), f32),
        scratch_shapes=[
            pltpu.VMEM((_PAD, _BLK), jnp.bfloat16),
            pltpu.VMEM((1, _PAD), f32),
            pltpu.VMEM((1, _PAD), f32),
        ],
    )(bt, d)
    return out[:_POST, :5]
